# Initial kernel scaffold; baseline (speedup 1.0000x reference)
import jax
import jax.numpy as jnp
from jax.experimental import pallas as pl


def kernel(x, edge_index, W1, b1, W2, b2):
    raise NotImplementedError("write your pallas kernel here")


# speedup vs baseline: 33.3381x; 33.3381x over previous
import functools
import jax
import jax.numpy as jnp
from jax import lax
from jax.experimental import pallas as pl
from jax.experimental.pallas import tpu as pltpu
from jax.experimental.pallas import tpu_sc as plsc

NC, NS, L = 2, 16, 16
NW = NC * NS
N_NODES = 10000
HID = 16
CH = 128
NPAD = 10112
RPT = NPAD // NS

_mesh = plsc.VectorSubcoreMesh(core_axis_name="c", subcore_axis_name="s",
                               num_cores=NC, num_subcores=NS)


def _zero_rows(buf, nrows):
    def body(i, _):
        buf[i, :] = jnp.zeros((L,), jnp.float32)
        return 0
    lax.fori_loop(0, nrows, body, 0)


def _make_deg_kernel(nchunk):
    @functools.partial(
        pl.kernel,
        out_type=jax.ShapeDtypeStruct((NC, NPAD, L), jnp.float32),
        mesh=_mesh,
        scratch_types=[
            pltpu.VMEM((nchunk, CH), jnp.int32),
            pltpu.VMEM((CH, L), jnp.float32),
            pltpu.VMEM((RPT, L), jnp.float32),
            pltpu.VMEM_SHARED((NPAD, L), jnp.float32),
        ],
        compiler_params=pltpu.CompilerParams(use_tc_tiling_on_sc=False),
    )
    def deg_kernel(dst_hbm, out_hbm, dst_v, ones_v, zbuf, acc):
        c = lax.axis_index("c")
        s = lax.axis_index("s")
        wid = c * NS + s
        _zero_rows(zbuf, RPT)
        pltpu.sync_copy(zbuf, acc.at[pl.ds(s * RPT, RPT)])

        def fill_ones(i, _):
            ones_v[i, :] = jnp.ones((L,), jnp.float32)
            return 0
        lax.fori_loop(0, CH, fill_ones, 0)
        pltpu.sync_copy(dst_hbm.at[wid], dst_v)
        plsc.subcore_barrier()

        def chunk(j, _):
            pltpu.sync_copy(ones_v, acc.at[dst_v.at[j]], add=True)
            return 0
        lax.fori_loop(0, nchunk, chunk, 0)
        plsc.subcore_barrier()
        pltpu.sync_copy(acc.at[pl.ds(s * RPT, RPT)],
                        out_hbm.at[c, pl.ds(s * RPT, RPT)])

    return deg_kernel


def _make_agg_kernel(nchunk):
    @functools.partial(
        pl.kernel,
        out_type=jax.ShapeDtypeStruct((NC, NPAD, L), jnp.float32),
        mesh=_mesh,
        scratch_types=[
            pltpu.VMEM((nchunk, CH), jnp.int32),
            pltpu.VMEM((nchunk, CH), jnp.int32),
            pltpu.VMEM((CH, L), jnp.float32),
            pltpu.VMEM((RPT, L), jnp.float32),
            pltpu.VMEM_SHARED((NPAD, L), jnp.float32),
            pltpu.SemaphoreType.DMA,
        ],
        compiler_params=pltpu.CompilerParams(use_tc_tiling_on_sc=False),
    )
    def agg_kernel(table_hbm, src_hbm, dst_hbm, out_hbm,
                   src_v, dst_v, gbuf, zbuf, acc, sem):
        c = lax.axis_index("c")
        s = lax.axis_index("s")
        wid = c * NS + s
        _zero_rows(zbuf, RPT)
        pltpu.sync_copy(zbuf, acc.at[pl.ds(s * RPT, RPT)])
        pltpu.sync_copy(src_hbm.at[wid], src_v)
        pltpu.sync_copy(dst_hbm.at[wid], dst_v)
        plsc.subcore_barrier()

        def chunk(j, _):
            pltpu.async_copy(table_hbm.at[src_v.at[j]], gbuf, sem).wait()
            pltpu.sync_copy(gbuf, acc.at[dst_v.at[j]], add=True)
            return 0
        lax.fori_loop(0, nchunk, chunk, 0)
        plsc.subcore_barrier()
        pltpu.sync_copy(acc.at[pl.ds(s * RPT, RPT)],
                        out_hbm.at[c, pl.ds(s * RPT, RPT)])

    return agg_kernel


def _matmul_scale_kernel(x_ref, w_ref, deg_ref, g_ref, dis_ref):
    h = jnp.dot(x_ref[...], w_ref[...], preferred_element_type=jnp.float32)
    cnt = deg_ref[0] + deg_ref[1]
    dis = lax.rsqrt(cnt + 1.0)
    g_ref[...] = dis * h
    dis_ref[...] = dis


def _mid_kernel(p_ref, g_ref, dis_ref, b_ref, q_ref):
    s = p_ref[0] + p_ref[1]
    a = dis_ref[...] * (s + g_ref[...]) + b_ref[...]
    q_ref[...] = dis_ref[...] * jnp.maximum(a, 0.0)


def _final_kernel(p_ref, q_ref, dis_ref, w_ref, b_ref, o_ref):
    s = p_ref[0] + p_ref[1]
    o = dis_ref[...] * (s + q_ref[...])
    o_ref[...] = (jnp.dot(o, w_ref[...], preferred_element_type=jnp.float32)
                  + b_ref[...])


def kernel(x, edge_index, W1, b1, W2, b2):
    n, f_in = x.shape
    hid = W1.shape[1]
    c_out = W2.shape[1]
    e = edge_index.shape[1]

    per_dma = NW * CH
    nchunk = -(-e // per_dma)
    ep = nchunk * per_dma
    src = edge_index[0]
    dst = edge_index[1]
    padv = jnp.full((ep - e,), N_NODES, jnp.int32)
    srcw = jnp.concatenate([src, padv]).reshape(NW, nchunk, CH)
    dstw = jnp.concatenate([dst, padv]).reshape(NW, nchunk, CH)

    kpad = -f_in % 128
    xp = jnp.pad(x, ((0, 0), (0, kpad)))
    w1p = jnp.pad(W1, ((0, kpad), (0, 0)))

    deg_k = _make_deg_kernel(nchunk)
    agg_k = _make_agg_kernel(nchunk)

    deg_parts = deg_k(dstw)

    mblk = 1000
    grid = (n // mblk,)
    g1, dis_t = pl.pallas_call(
        _matmul_scale_kernel,
        grid=grid,
        in_specs=[
            pl.BlockSpec((mblk, f_in + kpad), lambda i: (i, 0)),
            pl.BlockSpec((f_in + kpad, hid), lambda i: (0, 0)),
            pl.BlockSpec((NC, mblk, L), lambda i: (0, i, 0)),
        ],
        out_specs=[
            pl.BlockSpec((mblk, hid), lambda i: (i, 0)),
            pl.BlockSpec((mblk, L), lambda i: (i, 0)),
        ],
        out_shape=[
            jax.ShapeDtypeStruct((n, hid), jnp.float32),
            jax.ShapeDtypeStruct((n, L), jnp.float32),
        ],
    )(xp, w1p, deg_parts[:, :n, :])

    g1p = jnp.pad(g1, ((0, NPAD - n), (0, 0)))

    s1_parts = agg_k(g1p, srcw, dstw)

    q = pl.pallas_call(
        _mid_kernel,
        out_shape=jax.ShapeDtypeStruct((n, hid), jnp.float32),
    )(s1_parts[:, :n, :], g1, dis_t, jnp.broadcast_to(b1, (1, hid)))

    qp = jnp.pad(q, ((0, NPAD - n), (0, 0)))

    s2_parts = agg_k(qp, srcw, dstw)

    cpad = -c_out % 8
    w2p = jnp.pad(W2, ((0, 0), (0, cpad)))
    b2p = jnp.pad(b2, (0, cpad))
    outp = pl.pallas_call(
        _final_kernel,
        out_shape=jax.ShapeDtypeStruct((n, c_out + cpad), jnp.float32),
    )(s2_parts[:, :n, :], q, dis_t, w2p, jnp.broadcast_to(b2p, (1, c_out + cpad)))
    return outp[:, :c_out]


# speedup vs baseline: 55.2544x; 1.6574x over previous
import functools
import jax
import jax.numpy as jnp
from jax import lax
from jax.experimental import pallas as pl
from jax.experimental.pallas import tpu as pltpu
from jax.experimental.pallas import tpu_sc as plsc

NC, NS, L = 2, 16, 16
NW = NC * NS
N_NODES = 10000
HID = 16
CH = 128
NPAD = 10112
RPT = NPAD // NS
NBUF = 4

_mesh = plsc.VectorSubcoreMesh(core_axis_name="c", subcore_axis_name="s",
                               num_cores=NC, num_subcores=NS)


def _zero_rows(buf, nrows):
    def body(i, _):
        buf[i, :] = jnp.zeros((L,), jnp.float32)
        return 0
    lax.fori_loop(0, nrows, body, 0)


def _make_deg_kernel(nchunk):
    @functools.partial(
        pl.kernel,
        out_type=jax.ShapeDtypeStruct((NC, NPAD, L), jnp.float32),
        mesh=_mesh,
        scratch_types=[
            pltpu.VMEM((nchunk, CH), jnp.int32),
            pltpu.VMEM((CH, L), jnp.float32),
            pltpu.VMEM((RPT, L), jnp.float32),
            pltpu.VMEM_SHARED((NPAD, L), jnp.float32),
            pltpu.SemaphoreType.DMA,
        ],
        compiler_params=pltpu.CompilerParams(use_tc_tiling_on_sc=False),
    )
    def deg_kernel(dst_hbm, out_hbm, dst_v, ones_v, zbuf, acc, sem):
        c = lax.axis_index("c")
        s = lax.axis_index("s")
        wid = c * NS + s
        _zero_rows(zbuf, RPT)
        pltpu.sync_copy(zbuf, acc.at[pl.ds(s * RPT, RPT)])

        def fill_ones(i, _):
            ones_v[i, :] = jnp.ones((L,), jnp.float32)
            return 0
        lax.fori_loop(0, CH, fill_ones, 0)
        pltpu.sync_copy(dst_hbm.at[wid], dst_v)
        plsc.subcore_barrier()

        def fire(j, _):
            pltpu.async_copy(ones_v, acc.at[dst_v.at[j]], sem, add=True)
            return 0
        lax.fori_loop(0, nchunk, fire, 0)

        def drain(j, _):
            pltpu.make_async_copy(ones_v, acc.at[dst_v.at[j]], sem).wait()
            return 0
        lax.fori_loop(0, nchunk, drain, 0)
        plsc.subcore_barrier()
        pltpu.sync_copy(acc.at[pl.ds(s * RPT, RPT)],
                        out_hbm.at[c, pl.ds(s * RPT, RPT)])

    return deg_kernel


def _make_agg_kernel(nchunk):
    @functools.partial(
        pl.kernel,
        out_type=jax.ShapeDtypeStruct((NC, NPAD, L), jnp.float32),
        mesh=_mesh,
        scratch_types=[
            pltpu.VMEM((nchunk, CH), jnp.int32),
            pltpu.VMEM((nchunk, CH), jnp.int32),
            pltpu.VMEM((NBUF, CH, L), jnp.float32),
            pltpu.VMEM((RPT, L), jnp.float32),
            pltpu.VMEM_SHARED((NPAD, L), jnp.float32),
            pltpu.VMEM_SHARED((NPAD, L), jnp.float32),
            pltpu.SemaphoreType.DMA((NBUF,)),
            pltpu.SemaphoreType.DMA((NBUF,)),
        ],
        compiler_params=pltpu.CompilerParams(use_tc_tiling_on_sc=False),
    )
    def agg_kernel(table_hbm, src_hbm, dst_hbm, out_hbm,
                   src_v, dst_v, gbuf, zbuf, tbl, acc, gsem, ssem):
        c = lax.axis_index("c")
        s = lax.axis_index("s")
        wid = c * NS + s
        _zero_rows(zbuf, RPT)
        pltpu.sync_copy(zbuf, acc.at[pl.ds(s * RPT, RPT)])
        pltpu.sync_copy(table_hbm.at[pl.ds(s * RPT, RPT)],
                        tbl.at[pl.ds(s * RPT, RPT)])
        pltpu.sync_copy(src_hbm.at[wid], src_v)
        pltpu.sync_copy(dst_hbm.at[wid], dst_v)
        plsc.subcore_barrier()

        def gissue(j, slot):
            pltpu.async_copy(tbl.at[src_v.at[j]], gbuf.at[slot],
                             gsem.at[slot])

        def gwait(j, slot):
            pltpu.make_async_copy(tbl.at[src_v.at[j]], gbuf.at[slot],
                                  gsem.at[slot]).wait()

        def sissue(j, slot):
            pltpu.async_copy(gbuf.at[slot], acc.at[dst_v.at[j]],
                             ssem.at[slot], add=True)

        def swait(j, slot):
            pltpu.make_async_copy(gbuf.at[slot], acc.at[dst_v.at[j]],
                                  ssem.at[slot]).wait()

        gissue(0, 0)
        gissue(1, 1)
        ngroups = (nchunk + NBUF - 1) // NBUF

        def group(g, _):
            base = g * NBUF
            for b in range(NBUF):
                j = base + b
                b2 = (b + 2) % NBUF

                @pl.when(j < nchunk)
                def _():
                    gwait(j, b)
                    sissue(j, b)

                    @pl.when(j >= 2)
                    def _():
                        swait(j - 2, b2)

                    @pl.when(j + 2 < nchunk)
                    def _():
                        gissue(j + 2, b2)
            return 0

        lax.fori_loop(0, ngroups, group, 0)
        swait(nchunk - 2, (nchunk - 2) % NBUF)
        swait(nchunk - 1, (nchunk - 1) % NBUF)
        plsc.subcore_barrier()
        pltpu.sync_copy(acc.at[pl.ds(s * RPT, RPT)],
                        out_hbm.at[c, pl.ds(s * RPT, RPT)])

    return agg_kernel


def _matmul_kernel(x_ref, w_ref, h_ref):
    h_ref[...] = jnp.dot(x_ref[...], w_ref[...],
                         preferred_element_type=jnp.float32)


def _scale_kernel(h_ref, deg_ref, g_ref, dis_ref):
    cnt = deg_ref[0] + deg_ref[1]
    dis = lax.rsqrt(cnt + 1.0)
    g_ref[...] = dis * h_ref[...]
    dis_ref[...] = dis


def _mid_kernel(p_ref, g_ref, dis_ref, b_ref, q_ref):
    s = p_ref[0] + p_ref[1]
    a = dis_ref[...] * (s + g_ref[...]) + b_ref[...]
    q_ref[...] = dis_ref[...] * jnp.maximum(a, 0.0)


def _final_kernel(p_ref, q_ref, dis_ref, w_ref, b_ref, o_ref):
    s = p_ref[0] + p_ref[1]
    o = dis_ref[...] * (s + q_ref[...])
    o_ref[...] = (jnp.dot(o, w_ref[...], preferred_element_type=jnp.float32)
                  + b_ref[...])


def kernel(x, edge_index, W1, b1, W2, b2):
    n, f_in = x.shape
    hid = W1.shape[1]
    c_out = W2.shape[1]
    e = edge_index.shape[1]

    per_dma = NW * CH
    nchunk = -(-e // per_dma)
    ep = nchunk * per_dma
    src = edge_index[0]
    dst = edge_index[1]
    padv = jnp.full((ep - e,), N_NODES, jnp.int32)
    srcw = jnp.concatenate([src, padv]).reshape(NW, nchunk, CH)
    dstw = jnp.concatenate([dst, padv]).reshape(NW, nchunk, CH)

    kpad = -f_in % 128
    xp = jnp.pad(x, ((0, 0), (0, kpad)))
    w1p = jnp.pad(W1, ((0, kpad), (0, 0)))

    deg_k = _make_deg_kernel(nchunk)
    agg_k = _make_agg_kernel(nchunk)

    deg_parts = deg_k(dstw)

    mblk = 1000
    grid = (n // mblk,)
    h = pl.pallas_call(
        _matmul_kernel,
        grid=grid,
        in_specs=[
            pl.BlockSpec((mblk, f_in + kpad), lambda i: (i, 0)),
            pl.BlockSpec((f_in + kpad, hid), lambda i: (0, 0)),
        ],
        out_specs=pl.BlockSpec((mblk, hid), lambda i: (i, 0)),
        out_shape=jax.ShapeDtypeStruct((n, hid), jnp.float32),
    )(xp, w1p)

    g1, dis_t = pl.pallas_call(
        _scale_kernel,
        out_shape=[
            jax.ShapeDtypeStruct((n, hid), jnp.float32),
            jax.ShapeDtypeStruct((n, L), jnp.float32),
        ],
    )(h, deg_parts[:, :n, :])

    g1p = jnp.pad(g1, ((0, NPAD - n), (0, 0)))

    s1_parts = agg_k(g1p, srcw, dstw)

    q = pl.pallas_call(
        _mid_kernel,
        out_shape=jax.ShapeDtypeStruct((n, hid), jnp.float32),
    )(s1_parts[:, :n, :], g1, dis_t, jnp.broadcast_to(b1, (1, hid)))

    qp = jnp.pad(q, ((0, NPAD - n), (0, 0)))

    s2_parts = agg_k(qp, srcw, dstw)

    cpad = -c_out % 8
    w2p = jnp.pad(W2, ((0, 0), (0, cpad)))
    b2p = jnp.pad(b2, (0, cpad))
    outp = pl.pallas_call(
        _final_kernel,
        out_shape=jax.ShapeDtypeStruct((n, c_out + cpad), jnp.float32),
    )(s2_parts[:, :n, :], q, dis_t, w2p, jnp.broadcast_to(b2p, (1, c_out + cpad)))
    return outp[:, :c_out]


# speedup vs baseline: 59.0954x; 1.0695x over previous
import functools
import jax
import jax.numpy as jnp
from jax import lax
from jax.experimental import pallas as pl
from jax.experimental.pallas import tpu as pltpu
from jax.experimental.pallas import tpu_sc as plsc

NC, NS, L = 2, 16, 16
NW = NC * NS
N_NODES = 10000
HID = 16
CH = 128
NPAD = 10112
RPT = NPAD // NS
NBUF = 4

_mesh = plsc.VectorSubcoreMesh(core_axis_name="c", subcore_axis_name="s",
                               num_cores=NC, num_subcores=NS)


def _zero_rows(buf, nrows):
    def body(i, _):
        buf[i, :] = jnp.zeros((L,), jnp.float32)
        return 0
    lax.fori_loop(0, nrows, body, 0)


def _rsqrt16(x):
    i = plsc.bitcast(x, jnp.int32)
    i = 0x5F3759DF - lax.shift_right_logical(i, 1)
    y = plsc.bitcast(i, jnp.float32)
    for _ in range(3):
        y = y * (1.5 - 0.5 * x * y * y)
    return y


def _make_deg_kernel(nchunk):
    @functools.partial(
        pl.kernel,
        out_type=jax.ShapeDtypeStruct((NC, NPAD, L), jnp.float32),
        mesh=_mesh,
        scratch_types=[
            pltpu.VMEM((nchunk, CH), jnp.int32),
            pltpu.VMEM((CH, L), jnp.float32),
            pltpu.VMEM((RPT, L), jnp.float32),
            pltpu.VMEM_SHARED((NPAD, L), jnp.float32),
            pltpu.SemaphoreType.DMA,
        ],
        compiler_params=pltpu.CompilerParams(use_tc_tiling_on_sc=False, needs_layout_passes=False),
    )
    def deg_kernel(dst_hbm, out_hbm, dst_v, ones_v, zbuf, acc, sem):
        c = lax.axis_index("c")
        s = lax.axis_index("s")
        wid = c * NS + s
        _zero_rows(zbuf, RPT)
        pltpu.sync_copy(zbuf, acc.at[pl.ds(s * RPT, RPT)])

        def fill_ones(i, _):
            ones_v[i, :] = jnp.ones((L,), jnp.float32)
            return 0
        lax.fori_loop(0, CH, fill_ones, 0)
        pltpu.sync_copy(dst_hbm.at[wid], dst_v)
        plsc.subcore_barrier()

        def fire(j, _):
            pltpu.async_copy(ones_v, acc.at[dst_v.at[j]], sem, add=True)
            return 0
        lax.fori_loop(0, nchunk, fire, 0)

        def drain(j, _):
            pltpu.make_async_copy(ones_v, acc.at[dst_v.at[j]], sem).wait()
            return 0
        lax.fori_loop(0, nchunk, drain, 0)
        plsc.subcore_barrier()
        pltpu.sync_copy(acc.at[pl.ds(s * RPT, RPT)],
                        out_hbm.at[c, pl.ds(s * RPT, RPT)])

    return deg_kernel


def _agg_core(nchunk, src_v, dst_v, gbuf, tbl, acc, gsem, ssem):
    def gissue(j, slot):
        pltpu.async_copy(tbl.at[src_v.at[j]], gbuf.at[slot], gsem.at[slot])

    def gwait(j, slot):
        pltpu.make_async_copy(tbl.at[src_v.at[j]], gbuf.at[slot],
                              gsem.at[slot]).wait()

    def sissue(j, slot):
        pltpu.async_copy(gbuf.at[slot], acc.at[dst_v.at[j]], ssem.at[slot],
                         add=True)

    def swait(j, slot):
        pltpu.make_async_copy(gbuf.at[slot], acc.at[dst_v.at[j]],
                              ssem.at[slot]).wait()

    gissue(0, 0)
    gissue(1, 1)
    ngroups = (nchunk + NBUF - 1) // NBUF

    def group(g, _):
        base = g * NBUF
        for b in range(NBUF):
            j = base + b
            b2 = (b + 2) % NBUF

            @pl.when(j < nchunk)
            def _():
                gwait(j, b)
                sissue(j, b)

                @pl.when(j >= 2)
                def _():
                    swait(j - 2, b2)

                @pl.when(j + 2 < nchunk)
                def _():
                    gissue(j + 2, b2)
        return 0

    lax.fori_loop(0, ngroups, group, 0)
    swait(nchunk - 2, (nchunk - 2) % NBUF)
    swait(nchunk - 1, (nchunk - 1) % NBUF)


def _make_agg1_kernel(nchunk):
    @functools.partial(
        pl.kernel,
        out_type=[
            jax.ShapeDtypeStruct((NC, NPAD, L), jnp.float32),
            jax.ShapeDtypeStruct((NPAD, L), jnp.float32),
            jax.ShapeDtypeStruct((NPAD, L), jnp.float32),
        ],
        mesh=_mesh,
        scratch_types=[
            pltpu.VMEM((nchunk, CH), jnp.int32),
            pltpu.VMEM((nchunk, CH), jnp.int32),
            pltpu.VMEM((NBUF, CH, L), jnp.float32),
            pltpu.VMEM((RPT, L), jnp.float32),
            pltpu.VMEM((RPT, L), jnp.float32),
            pltpu.VMEM((RPT, L), jnp.float32),
            pltpu.VMEM((RPT, L), jnp.float32),
            pltpu.VMEM_SHARED((NPAD, L), jnp.float32),
            pltpu.VMEM_SHARED((NPAD, L), jnp.float32),
            pltpu.SemaphoreType.DMA((NBUF,)),
            pltpu.SemaphoreType.DMA((NBUF,)),
        ],
        compiler_params=pltpu.CompilerParams(use_tc_tiling_on_sc=False, needs_layout_passes=False),
    )
    def agg1_kernel(h_hbm, deg_hbm, src_hbm, dst_hbm,
                    out_hbm, g1_hbm, dis_hbm,
                    src_v, dst_v, gbuf, hv, av, bv, zbuf, tbl, acc,
                    gsem, ssem):
        c = lax.axis_index("c")
        s = lax.axis_index("s")
        wid = c * NS + s
        r0 = s * RPT
        _zero_rows(zbuf, RPT)
        pltpu.sync_copy(zbuf, acc.at[pl.ds(r0, RPT)])
        pltpu.sync_copy(h_hbm.at[pl.ds(r0, RPT)], hv)
        pltpu.sync_copy(deg_hbm.at[0, pl.ds(r0, RPT)], av)
        pltpu.sync_copy(deg_hbm.at[1, pl.ds(r0, RPT)], bv)
        pltpu.sync_copy(src_hbm.at[wid], src_v)
        pltpu.sync_copy(dst_hbm.at[wid], dst_v)

        def build(i, _):
            cnt = av[i, :] + bv[i, :] + 1.0
            dis = _rsqrt16(cnt)
            av[i, :] = dis
            bv[i, :] = dis * hv[i, :]
            return 0
        lax.fori_loop(0, RPT, build, 0)
        pltpu.sync_copy(bv, tbl.at[pl.ds(r0, RPT)])

        @pl.when(c == 0)
        def _():
            pltpu.sync_copy(bv, g1_hbm.at[pl.ds(r0, RPT)])
            pltpu.sync_copy(av, dis_hbm.at[pl.ds(r0, RPT)])
        plsc.subcore_barrier()

        _agg_core(nchunk, src_v, dst_v, gbuf, tbl, acc, gsem, ssem)
        plsc.subcore_barrier()
        pltpu.sync_copy(acc.at[pl.ds(r0, RPT)],
                        out_hbm.at[c, pl.ds(r0, RPT)])

    return agg1_kernel


def _make_agg2_kernel(nchunk):
    @functools.partial(
        pl.kernel,
        out_type=[
            jax.ShapeDtypeStruct((NC, NPAD, L), jnp.float32),
            jax.ShapeDtypeStruct((NPAD, L), jnp.float32),
        ],
        mesh=_mesh,
        scratch_types=[
            pltpu.VMEM((nchunk, CH), jnp.int32),
            pltpu.VMEM((nchunk, CH), jnp.int32),
            pltpu.VMEM((NBUF, CH, L), jnp.float32),
            pltpu.VMEM((RPT, L), jnp.float32),
            pltpu.VMEM((RPT, L), jnp.float32),
            pltpu.VMEM((RPT, L), jnp.float32),
            pltpu.VMEM((RPT, L), jnp.float32),
            pltpu.VMEM((RPT, L), jnp.float32),
            pltpu.VMEM_SHARED((NPAD, L), jnp.float32),
            pltpu.VMEM_SHARED((NPAD, L), jnp.float32),
            pltpu.SemaphoreType.DMA((NBUF,)),
            pltpu.SemaphoreType.DMA((NBUF,)),
        ],
        compiler_params=pltpu.CompilerParams(use_tc_tiling_on_sc=False, needs_layout_passes=False),
    )
    def agg2_kernel(g1_hbm, p_hbm, dis_hbm, b1_hbm, src_hbm, dst_hbm,
                    out_hbm, q_hbm,
                    src_v, dst_v, gbuf, gv, av, bv, dv, zbuf, tbl, acc,
                    gsem, ssem):
        c = lax.axis_index("c")
        s = lax.axis_index("s")
        wid = c * NS + s
        r0 = s * RPT
        _zero_rows(zbuf, RPT)
        pltpu.sync_copy(zbuf, acc.at[pl.ds(r0, RPT)])
        pltpu.sync_copy(g1_hbm.at[pl.ds(r0, RPT)], gv)
        pltpu.sync_copy(p_hbm.at[0, pl.ds(r0, RPT)], av)
        pltpu.sync_copy(p_hbm.at[1, pl.ds(r0, RPT)], bv)
        pltpu.sync_copy(dis_hbm.at[pl.ds(r0, RPT)], dv)
        pltpu.sync_copy(b1_hbm, zbuf.at[pl.ds(0, 1)])
        pltpu.sync_copy(src_hbm.at[wid], src_v)
        pltpu.sync_copy(dst_hbm.at[wid], dst_v)
        b1 = zbuf[0, :]

        def build(i, _):
            dis = dv[i, :]
            a = dis * (av[i, :] + bv[i, :] + gv[i, :]) + b1
            bv[i, :] = dis * jnp.maximum(a, 0.0)
            return 0
        lax.fori_loop(0, RPT, build, 0)
        pltpu.sync_copy(bv, tbl.at[pl.ds(r0, RPT)])

        @pl.when(c == 0)
        def _():
            pltpu.sync_copy(bv, q_hbm.at[pl.ds(r0, RPT)])
        plsc.subcore_barrier()

        _agg_core(nchunk, src_v, dst_v, gbuf, tbl, acc, gsem, ssem)
        plsc.subcore_barrier()
        pltpu.sync_copy(acc.at[pl.ds(r0, RPT)],
                        out_hbm.at[c, pl.ds(r0, RPT)])

    return agg2_kernel


def _matmul_kernel(x_ref, w_ref, h_ref):
    h_ref[...] = jnp.dot(x_ref[...], w_ref[...],
                         preferred_element_type=jnp.float32)


def _final_kernel(p_ref, q_ref, dis_ref, w_ref, b_ref, o_ref):
    s = p_ref[0] + p_ref[1]
    o = dis_ref[...] * (s + q_ref[...])
    o_ref[...] = (jnp.dot(o, w_ref[...], preferred_element_type=jnp.float32)
                  + b_ref[...])


def kernel(x, edge_index, W1, b1, W2, b2):
    n, f_in = x.shape
    hid = W1.shape[1]
    c_out = W2.shape[1]
    e = edge_index.shape[1]

    per_dma = NW * CH
    nchunk = -(-e // per_dma)
    ep = nchunk * per_dma
    src = edge_index[0]
    dst = edge_index[1]
    padv = (n + jnp.arange(ep - e, dtype=jnp.int32) % (NPAD - n)
            ).astype(jnp.int32)
    srcw = jnp.concatenate([src, padv]).reshape(NW, nchunk, CH)
    dstw = jnp.concatenate([dst, padv]).reshape(NW, nchunk, CH)

    kpad = -f_in % 128
    xp = jnp.pad(x, ((0, 0), (0, kpad)))
    w1p = jnp.pad(W1, ((0, kpad), (0, 0)))

    deg_k = _make_deg_kernel(nchunk)
    agg1_k = _make_agg1_kernel(nchunk)
    agg2_k = _make_agg2_kernel(nchunk)

    deg_parts = deg_k(dstw)

    mblk = 1000
    grid = (n // mblk,)
    h = pl.pallas_call(
        _matmul_kernel,
        grid=grid,
        in_specs=[
            pl.BlockSpec((mblk, f_in + kpad), lambda i: (i, 0)),
            pl.BlockSpec((f_in + kpad, hid), lambda i: (0, 0)),
        ],
        out_specs=pl.BlockSpec((mblk, hid), lambda i: (i, 0)),
        out_shape=jax.ShapeDtypeStruct((n, hid), jnp.float32),
    )(xp, w1p)
    hp = jnp.pad(h, ((0, NPAD - n), (0, 0)))

    s1_parts, g1p, dis_tp = agg1_k(hp, deg_parts, srcw, dstw)

    s2_parts, qp = agg2_k(g1p, s1_parts, dis_tp,
                          jnp.broadcast_to(b1, (1, hid)), srcw, dstw)

    cpad = -c_out % 8
    w2p = jnp.pad(W2, ((0, 0), (0, cpad)))
    b2p = jnp.pad(b2, (0, cpad))
    outp = pl.pallas_call(
        _final_kernel,
        out_shape=jax.ShapeDtypeStruct((n, c_out + cpad), jnp.float32),
    )(s2_parts[:, :n, :], qp[:n], dis_tp[:n],
      w2p, jnp.broadcast_to(b2p, (1, c_out + cpad)))
    return outp[:, :c_out]


# speedup vs baseline: 86.8526x; 1.4697x over previous
import functools
import jax
import jax.numpy as jnp
from jax import lax
from jax.experimental import pallas as pl
from jax.experimental.pallas import tpu as pltpu
from jax.experimental.pallas import tpu_sc as plsc

NC, NS, L = 2, 16, 16
NW = NC * NS
N_NODES = 10000
HID = 16
CH = 128
NPAD = 10112
RPT = NPAD // NS
NBUF = 4

_mesh = plsc.VectorSubcoreMesh(core_axis_name="c", subcore_axis_name="s",
                               num_cores=NC, num_subcores=NS)


def _zero_rows(buf, nrows):
    def body(i, _):
        buf[i, :] = jnp.zeros((L,), jnp.float32)
        return 0
    lax.fori_loop(0, nrows, body, 0)


def _rsqrt16(x):
    i = plsc.bitcast(x, jnp.int32)
    i = 0x5F3759DF - lax.shift_right_logical(i, 1)
    y = plsc.bitcast(i, jnp.float32)
    for _ in range(3):
        y = y * (1.5 - 0.5 * x * y * y)
    return y


def _make_deg_kernel(nchunk):
    @functools.partial(
        pl.kernel,
        out_type=jax.ShapeDtypeStruct((NC, NPAD, L), jnp.float32),
        mesh=_mesh,
        scratch_types=[
            pltpu.VMEM((nchunk, CH), jnp.int32),
            pltpu.VMEM((CH, L), jnp.float32),
            pltpu.VMEM((RPT, L), jnp.float32),
            pltpu.VMEM_SHARED((NPAD, L), jnp.float32),
            pltpu.SemaphoreType.DMA,
        ],
        compiler_params=pltpu.CompilerParams(use_tc_tiling_on_sc=False, needs_layout_passes=False),
    )
    def deg_kernel(dst_hbm, out_hbm, dst_v, ones_v, zbuf, acc, sem):
        c = lax.axis_index("c")
        s = lax.axis_index("s")
        wid = c * NS + s
        _zero_rows(zbuf, RPT)
        pltpu.sync_copy(zbuf, acc.at[pl.ds(s * RPT, RPT)])

        def fill_ones(i, _):
            ones_v[i, :] = jnp.ones((L,), jnp.float32)
            return 0
        lax.fori_loop(0, CH, fill_ones, 0)
        pltpu.sync_copy(dst_hbm.at[wid], dst_v)
        plsc.subcore_barrier()

        def fire(j, _):
            pltpu.async_copy(ones_v, acc.at[dst_v.at[j]], sem, add=True)
            return 0
        lax.fori_loop(0, nchunk, fire, 0)

        def drain(j, _):
            pltpu.make_async_copy(ones_v, acc.at[dst_v.at[j]], sem).wait()
            return 0
        lax.fori_loop(0, nchunk, drain, 0)
        plsc.subcore_barrier()
        pltpu.sync_copy(acc.at[pl.ds(s * RPT, RPT)],
                        out_hbm.at[c, pl.ds(s * RPT, RPT)])

    return deg_kernel


def _agg_core(nchunk, src_v, dst_v, gbuf, tbl, acc, gsem, ssem):
    def gissue(j, slot):
        pltpu.async_copy(tbl.at[src_v.at[j]], gbuf.at[slot], gsem.at[slot])

    def gwait(j, slot):
        pltpu.make_async_copy(tbl.at[src_v.at[j]], gbuf.at[slot],
                              gsem.at[slot]).wait()

    def sissue(j, slot):
        pltpu.async_copy(gbuf.at[slot], acc.at[dst_v.at[j]], ssem.at[slot],
                         add=True)

    def swait(j, slot):
        pltpu.make_async_copy(gbuf.at[slot], acc.at[dst_v.at[j]],
                              ssem.at[slot]).wait()

    gissue(0, 0)
    gissue(1, 1)
    ngroups = (nchunk + NBUF - 1) // NBUF

    def group(g, _):
        base = g * NBUF
        for b in range(NBUF):
            j = base + b
            b2 = (b + 2) % NBUF

            @pl.when(j < nchunk)
            def _():
                gwait(j, b)
                sissue(j, b)

                @pl.when(j >= 2)
                def _():
                    swait(j - 2, b2)

                @pl.when(j + 2 < nchunk)
                def _():
                    gissue(j + 2, b2)
        return 0

    lax.fori_loop(0, ngroups, group, 0)
    swait(nchunk - 2, (nchunk - 2) % NBUF)
    swait(nchunk - 1, (nchunk - 1) % NBUF)


def _make_agg1_kernel(nchunk):
    @functools.partial(
        pl.kernel,
        out_type=[
            jax.ShapeDtypeStruct((NC, NPAD, L), jnp.float32),
            jax.ShapeDtypeStruct((NPAD, L), jnp.float32),
            jax.ShapeDtypeStruct((NPAD, L), jnp.float32),
        ],
        mesh=_mesh,
        scratch_types=[
            pltpu.VMEM((nchunk, CH), jnp.int32),
            pltpu.VMEM((nchunk, CH), jnp.int32),
            pltpu.VMEM((NBUF, CH, L), jnp.float32),
            pltpu.VMEM((RPT, L), jnp.float32),
            pltpu.VMEM((RPT, L), jnp.float32),
            pltpu.VMEM((RPT, L), jnp.float32),
            pltpu.VMEM((RPT, L), jnp.float32),
            pltpu.VMEM_SHARED((NPAD, L), jnp.float32),
            pltpu.VMEM_SHARED((NPAD, L), jnp.float32),
            pltpu.SemaphoreType.DMA((NBUF,)),
            pltpu.SemaphoreType.DMA((NBUF,)),
        ],
        compiler_params=pltpu.CompilerParams(use_tc_tiling_on_sc=False, needs_layout_passes=False),
    )
    def agg1_kernel(h_hbm, deg_hbm, src_hbm, dst_hbm,
                    out_hbm, g1_hbm, dis_hbm,
                    src_v, dst_v, gbuf, hv, av, bv, zbuf, tbl, acc,
                    gsem, ssem):
        c = lax.axis_index("c")
        s = lax.axis_index("s")
        wid = c * NS + s
        r0 = s * RPT
        _zero_rows(zbuf, RPT)
        pltpu.sync_copy(zbuf, acc.at[pl.ds(r0, RPT)])
        pltpu.sync_copy(h_hbm.at[pl.ds(r0, RPT)], hv)
        pltpu.sync_copy(deg_hbm.at[0, pl.ds(r0, RPT)], av)
        pltpu.sync_copy(deg_hbm.at[1, pl.ds(r0, RPT)], bv)
        pltpu.sync_copy(src_hbm.at[wid], src_v)
        pltpu.sync_copy(dst_hbm.at[wid], dst_v)

        def build(i, _):
            cnt = av[i, :] + bv[i, :] + 1.0
            dis = _rsqrt16(cnt)
            av[i, :] = dis
            bv[i, :] = dis * hv[i, :]
            return 0
        lax.fori_loop(0, RPT, build, 0)
        pltpu.sync_copy(bv, tbl.at[pl.ds(r0, RPT)])

        @pl.when(c == 0)
        def _():
            pltpu.sync_copy(bv, g1_hbm.at[pl.ds(r0, RPT)])
            pltpu.sync_copy(av, dis_hbm.at[pl.ds(r0, RPT)])
        plsc.subcore_barrier()

        _agg_core(nchunk, src_v, dst_v, gbuf, tbl, acc, gsem, ssem)
        plsc.subcore_barrier()
        pltpu.sync_copy(acc.at[pl.ds(r0, RPT)],
                        out_hbm.at[c, pl.ds(r0, RPT)])

    return agg1_kernel


def _make_agg2_kernel(nchunk):
    @functools.partial(
        pl.kernel,
        out_type=[
            jax.ShapeDtypeStruct((NC, NPAD, L), jnp.float32),
            jax.ShapeDtypeStruct((NPAD, L), jnp.float32),
        ],
        mesh=_mesh,
        scratch_types=[
            pltpu.VMEM((nchunk, CH), jnp.int32),
            pltpu.VMEM((nchunk, CH), jnp.int32),
            pltpu.VMEM((NBUF, CH, L), jnp.float32),
            pltpu.VMEM((RPT, L), jnp.float32),
            pltpu.VMEM((RPT, L), jnp.float32),
            pltpu.VMEM((RPT, L), jnp.float32),
            pltpu.VMEM((RPT, L), jnp.float32),
            pltpu.VMEM((RPT, L), jnp.float32),
            pltpu.VMEM_SHARED((NPAD, L), jnp.float32),
            pltpu.VMEM_SHARED((NPAD, L), jnp.float32),
            pltpu.SemaphoreType.DMA((NBUF,)),
            pltpu.SemaphoreType.DMA((NBUF,)),
        ],
        compiler_params=pltpu.CompilerParams(use_tc_tiling_on_sc=False, needs_layout_passes=False),
    )
    def agg2_kernel(g1_hbm, p_hbm, dis_hbm, b1_hbm, src_hbm, dst_hbm,
                    out_hbm, q_hbm,
                    src_v, dst_v, gbuf, gv, av, bv, dv, zbuf, tbl, acc,
                    gsem, ssem):
        c = lax.axis_index("c")
        s = lax.axis_index("s")
        wid = c * NS + s
        r0 = s * RPT
        _zero_rows(zbuf, RPT)
        pltpu.sync_copy(zbuf, acc.at[pl.ds(r0, RPT)])
        pltpu.sync_copy(g1_hbm.at[pl.ds(r0, RPT)], gv)
        pltpu.sync_copy(p_hbm.at[0, pl.ds(r0, RPT)], av)
        pltpu.sync_copy(p_hbm.at[1, pl.ds(r0, RPT)], bv)
        pltpu.sync_copy(dis_hbm.at[pl.ds(r0, RPT)], dv)
        pltpu.sync_copy(b1_hbm, zbuf.at[pl.ds(0, 1)])
        pltpu.sync_copy(src_hbm.at[wid], src_v)
        pltpu.sync_copy(dst_hbm.at[wid], dst_v)
        b1 = zbuf[0, :]

        def build(i, _):
            dis = dv[i, :]
            a = dis * (av[i, :] + bv[i, :] + gv[i, :]) + b1
            bv[i, :] = dis * jnp.maximum(a, 0.0)
            return 0
        lax.fori_loop(0, RPT, build, 0)
        pltpu.sync_copy(bv, tbl.at[pl.ds(r0, RPT)])

        @pl.when(c == 0)
        def _():
            pltpu.sync_copy(bv, q_hbm.at[pl.ds(r0, RPT)])
        plsc.subcore_barrier()

        _agg_core(nchunk, src_v, dst_v, gbuf, tbl, acc, gsem, ssem)
        plsc.subcore_barrier()
        pltpu.sync_copy(acc.at[pl.ds(r0, RPT)],
                        out_hbm.at[c, pl.ds(r0, RPT)])

    return agg2_kernel


def _matmul_kernel(x_ref, w_ref, h_ref):
    h_ref[...] = jnp.dot(x_ref[...], w_ref[...],
                         preferred_element_type=jnp.float32)


def _final_kernel(p_ref, q_ref, dis_ref, w_ref, b_ref, o_ref):
    n = o_ref.shape[0]
    s = p_ref[0, :n, :] + p_ref[1, :n, :]
    o = dis_ref[:n, :] * (s + q_ref[:n, :])
    o_ref[...] = (jnp.dot(o, w_ref[...], preferred_element_type=jnp.float32)
                  + b_ref[...])


def kernel(x, edge_index, W1, b1, W2, b2):
    n, f_in = x.shape
    hid = W1.shape[1]
    c_out = W2.shape[1]
    e = edge_index.shape[1]

    per_dma = NW * CH
    nchunk = -(-e // per_dma)
    ep = nchunk * per_dma
    src = edge_index[0]
    dst = edge_index[1]
    padv = (n + jnp.arange(ep - e, dtype=jnp.int32) % (NPAD - n)
            ).astype(jnp.int32)
    srcw = jnp.concatenate([src, padv]).reshape(NW, nchunk, CH)
    dstw = jnp.concatenate([dst, padv]).reshape(NW, nchunk, CH)

    deg_k = _make_deg_kernel(nchunk)
    agg1_k = _make_agg1_kernel(nchunk)
    agg2_k = _make_agg2_kernel(nchunk)

    deg_parts = deg_k(dstw)

    mblk = 1000
    grid = (n // mblk,)
    h = pl.pallas_call(
        _matmul_kernel,
        grid=grid,
        in_specs=[
            pl.BlockSpec((mblk, f_in), lambda i: (i, 0)),
            pl.BlockSpec((f_in, hid), lambda i: (0, 0)),
        ],
        out_specs=pl.BlockSpec((mblk, hid), lambda i: (i, 0)),
        out_shape=jax.ShapeDtypeStruct((n, hid), jnp.float32),
    )(x, W1)
    hp = jnp.pad(h, ((0, NPAD - n), (0, 0)))

    s1_parts, g1p, dis_tp = agg1_k(hp, deg_parts, srcw, dstw)

    s2_parts, qp = agg2_k(g1p, s1_parts, dis_tp,
                          jnp.broadcast_to(b1, (1, hid)), srcw, dstw)

    cpad = -c_out % 8
    w2p = jnp.pad(W2, ((0, 0), (0, cpad)))
    b2p = jnp.pad(b2, (0, cpad))
    outp = pl.pallas_call(
        _final_kernel,
        out_shape=jax.ShapeDtypeStruct((n, c_out + cpad), jnp.float32),
    )(s2_parts, qp, dis_tp, w2p, jnp.broadcast_to(b2p, (1, c_out + cpad)))
    return outp[:, :c_out]


# speedup vs baseline: 100.5001x; 1.1571x over previous
import functools
import jax
import jax.numpy as jnp
from jax import lax
from jax.experimental import pallas as pl
from jax.experimental.pallas import tpu as pltpu
from jax.experimental.pallas import tpu_sc as plsc

NC, NS, L = 2, 16, 16
NW = NC * NS
N_NODES = 10000
HID = 16
CH = 128
NPAD = 10112
RPT = NPAD // NS
NBUF = 8

_mesh = plsc.VectorSubcoreMesh(core_axis_name="c", subcore_axis_name="s",
                               num_cores=NC, num_subcores=NS)


def _zero_rows(buf, nrows):
    def body(i, _):
        buf[i, :] = jnp.zeros((L,), jnp.float32)
        return 0
    lax.fori_loop(0, nrows, body, 0)


def _rsqrt16(x):
    i = plsc.bitcast(x, jnp.int32)
    i = 0x5F3759DF - lax.shift_right_logical(i, 1)
    y = plsc.bitcast(i, jnp.float32)
    for _ in range(3):
        y = y * (1.5 - 0.5 * x * y * y)
    return y


def _make_deg_kernel(nchunk):
    @functools.partial(
        pl.kernel,
        out_type=jax.ShapeDtypeStruct((NC, NPAD, L), jnp.float32),
        mesh=_mesh,
        scratch_types=[
            pltpu.VMEM((nchunk, CH), jnp.int32),
            pltpu.VMEM((CH, L), jnp.float32),
            pltpu.VMEM((RPT, L), jnp.float32),
            pltpu.VMEM_SHARED((NPAD, L), jnp.float32),
            pltpu.SemaphoreType.DMA,
        ],
        compiler_params=pltpu.CompilerParams(use_tc_tiling_on_sc=False, needs_layout_passes=False),
    )
    def deg_kernel(dst_hbm, out_hbm, dst_v, ones_v, zbuf, acc, sem):
        c = lax.axis_index("c")
        s = lax.axis_index("s")
        wid = c * NS + s
        _zero_rows(zbuf, RPT)
        pltpu.sync_copy(zbuf, acc.at[pl.ds(s * RPT, RPT)])

        def fill_ones(i, _):
            ones_v[i, :] = jnp.ones((L,), jnp.float32)
            return 0
        lax.fori_loop(0, CH, fill_ones, 0)
        pltpu.sync_copy(dst_hbm.at[wid], dst_v)
        plsc.subcore_barrier()

        def fire(j, _):
            pltpu.async_copy(ones_v, acc.at[dst_v.at[j]], sem, add=True)
            return 0
        lax.fori_loop(0, nchunk, fire, 0)

        def drain(j, _):
            pltpu.make_async_copy(ones_v, acc.at[dst_v.at[j]], sem).wait()
            return 0
        lax.fori_loop(0, nchunk, drain, 0)
        plsc.subcore_barrier()
        pltpu.sync_copy(acc.at[pl.ds(s * RPT, RPT)],
                        out_hbm.at[c, pl.ds(s * RPT, RPT)])

    return deg_kernel


def _agg_core(nchunk, src_v, dst_v, gbuf, tbl, acc, gsem, ssem):
    def gissue(j, slot):
        pltpu.async_copy(tbl.at[src_v.at[j]], gbuf.at[slot], gsem.at[slot])

    def gwait(j, slot):
        pltpu.make_async_copy(tbl.at[src_v.at[j]], gbuf.at[slot],
                              gsem.at[slot]).wait()

    def sissue(j, slot):
        pltpu.async_copy(gbuf.at[slot], acc.at[dst_v.at[j]], ssem.at[slot],
                         add=True)

    def swait(j, slot):
        pltpu.make_async_copy(gbuf.at[slot], acc.at[dst_v.at[j]],
                              ssem.at[slot]).wait()

    la = NBUF // 2
    for j0 in range(la):
        gissue(j0, j0)
    ngroups = (nchunk + NBUF - 1) // NBUF

    def group(g, _):
        base = g * NBUF
        for b in range(NBUF):
            j = base + b
            b2 = (b + la) % NBUF

            @pl.when(j < nchunk)
            def _():
                gwait(j, b)
                sissue(j, b)

                @pl.when(j >= la)
                def _():
                    swait(j - la, b2)

                @pl.when(j + la < nchunk)
                def _():
                    gissue(j + la, b2)
        return 0

    lax.fori_loop(0, ngroups, group, 0)
    for k in range(la):
        j = nchunk - la + k
        swait(j, j % NBUF)


def _make_agg1_kernel(nchunk):
    @functools.partial(
        pl.kernel,
        out_type=[
            jax.ShapeDtypeStruct((NC, NPAD, L), jnp.float32),
            jax.ShapeDtypeStruct((NPAD, L), jnp.float32),
            jax.ShapeDtypeStruct((NPAD, L), jnp.float32),
        ],
        mesh=_mesh,
        scratch_types=[
            pltpu.VMEM((nchunk, CH), jnp.int32),
            pltpu.VMEM((nchunk, CH), jnp.int32),
            pltpu.VMEM((NBUF, CH, L), jnp.float32),
            pltpu.VMEM((RPT, L), jnp.float32),
            pltpu.VMEM((RPT, L), jnp.float32),
            pltpu.VMEM((RPT, L), jnp.float32),
            pltpu.VMEM((RPT, L), jnp.float32),
            pltpu.VMEM_SHARED((NPAD, L), jnp.float32),
            pltpu.VMEM_SHARED((NPAD, L), jnp.float32),
            pltpu.SemaphoreType.DMA((NBUF,)),
            pltpu.SemaphoreType.DMA((NBUF,)),
        ],
        compiler_params=pltpu.CompilerParams(use_tc_tiling_on_sc=False, needs_layout_passes=False),
    )
    def agg1_kernel(h_hbm, deg_hbm, src_hbm, dst_hbm,
                    out_hbm, g1_hbm, dis_hbm,
                    src_v, dst_v, gbuf, hv, av, bv, zbuf, tbl, acc,
                    gsem, ssem):
        c = lax.axis_index("c")
        s = lax.axis_index("s")
        wid = c * NS + s
        r0 = s * RPT
        _zero_rows(zbuf, RPT)
        c0 = pltpu.async_copy(zbuf, acc.at[pl.ds(r0, RPT)], ssem.at[0])
        c1 = pltpu.async_copy(h_hbm.at[pl.ds(r0, RPT)], hv, gsem.at[0])
        c2 = pltpu.async_copy(deg_hbm.at[0, pl.ds(r0, RPT)], av, gsem.at[1])
        c3 = pltpu.async_copy(deg_hbm.at[1, pl.ds(r0, RPT)], bv, gsem.at[2])
        c4 = pltpu.async_copy(src_hbm.at[wid], src_v, gsem.at[3])
        c5 = pltpu.async_copy(dst_hbm.at[wid], dst_v, ssem.at[1])
        c1.wait()
        c2.wait()
        c3.wait()

        def build(i, _):
            for u in range(4):
                r = i * 4 + u
                cnt = av[r, :] + bv[r, :] + 1.0
                dis = _rsqrt16(cnt)
                av[r, :] = dis
                bv[r, :] = dis * hv[r, :]
            return 0
        lax.fori_loop(0, RPT // 4, build, 0)
        pltpu.sync_copy(bv, tbl.at[pl.ds(r0, RPT)])

        @pl.when(c == 0)
        def _():
            pltpu.sync_copy(bv, g1_hbm.at[pl.ds(r0, RPT)])
            pltpu.sync_copy(av, dis_hbm.at[pl.ds(r0, RPT)])
        c0.wait()
        c4.wait()
        c5.wait()
        plsc.subcore_barrier()

        _agg_core(nchunk, src_v, dst_v, gbuf, tbl, acc, gsem, ssem)
        plsc.subcore_barrier()
        pltpu.sync_copy(acc.at[pl.ds(r0, RPT)],
                        out_hbm.at[c, pl.ds(r0, RPT)])

    return agg1_kernel


def _make_agg2_kernel(nchunk):
    @functools.partial(
        pl.kernel,
        out_type=[
            jax.ShapeDtypeStruct((NC, NPAD, L), jnp.float32),
            jax.ShapeDtypeStruct((NPAD, L), jnp.float32),
        ],
        mesh=_mesh,
        scratch_types=[
            pltpu.VMEM((nchunk, CH), jnp.int32),
            pltpu.VMEM((nchunk, CH), jnp.int32),
            pltpu.VMEM((NBUF, CH, L), jnp.float32),
            pltpu.VMEM((RPT, L), jnp.float32),
            pltpu.VMEM((RPT, L), jnp.float32),
            pltpu.VMEM((RPT, L), jnp.float32),
            pltpu.VMEM((RPT, L), jnp.float32),
            pltpu.VMEM((RPT, L), jnp.float32),
            pltpu.VMEM_SHARED((NPAD, L), jnp.float32),
            pltpu.VMEM_SHARED((NPAD, L), jnp.float32),
            pltpu.SemaphoreType.DMA((NBUF,)),
            pltpu.SemaphoreType.DMA((NBUF,)),
        ],
        compiler_params=pltpu.CompilerParams(use_tc_tiling_on_sc=False, needs_layout_passes=False),
    )
    def agg2_kernel(g1_hbm, p_hbm, dis_hbm, b1_hbm, src_hbm, dst_hbm,
                    out_hbm, q_hbm,
                    src_v, dst_v, gbuf, gv, av, bv, dv, zbuf, tbl, acc,
                    gsem, ssem):
        c = lax.axis_index("c")
        s = lax.axis_index("s")
        wid = c * NS + s
        r0 = s * RPT
        _zero_rows(zbuf, RPT)
        c0 = pltpu.async_copy(zbuf, acc.at[pl.ds(r0, RPT)], ssem.at[0])
        c1 = pltpu.async_copy(g1_hbm.at[pl.ds(r0, RPT)], gv, gsem.at[0])
        c2 = pltpu.async_copy(p_hbm.at[0, pl.ds(r0, RPT)], av, gsem.at[1])
        c3 = pltpu.async_copy(p_hbm.at[1, pl.ds(r0, RPT)], bv, gsem.at[2])
        c4 = pltpu.async_copy(dis_hbm.at[pl.ds(r0, RPT)], dv, gsem.at[3])
        c5 = pltpu.async_copy(src_hbm.at[wid], src_v, ssem.at[1])
        c6 = pltpu.async_copy(dst_hbm.at[wid], dst_v, ssem.at[2])
        pltpu.sync_copy(b1_hbm, zbuf.at[pl.ds(0, 1)])
        b1 = zbuf[0, :]
        c1.wait()
        c2.wait()
        c3.wait()
        c4.wait()

        def build(i, _):
            for u in range(4):
                r = i * 4 + u
                dis = dv[r, :]
                a = dis * (av[r, :] + bv[r, :] + gv[r, :]) + b1
                bv[r, :] = dis * jnp.maximum(a, 0.0)
            return 0
        lax.fori_loop(0, RPT // 4, build, 0)
        pltpu.sync_copy(bv, tbl.at[pl.ds(r0, RPT)])

        @pl.when(c == 0)
        def _():
            pltpu.sync_copy(bv, q_hbm.at[pl.ds(r0, RPT)])
        c0.wait()
        c5.wait()
        c6.wait()
        plsc.subcore_barrier()

        _agg_core(nchunk, src_v, dst_v, gbuf, tbl, acc, gsem, ssem)
        plsc.subcore_barrier()
        pltpu.sync_copy(acc.at[pl.ds(r0, RPT)],
                        out_hbm.at[c, pl.ds(r0, RPT)])

    return agg2_kernel


def _matmul_kernel(x_ref, w_ref, h_ref):
    h_ref[...] = jnp.dot(x_ref[...], w_ref[...],
                         preferred_element_type=jnp.float32)


def _final_kernel(p_ref, q_ref, dis_ref, w_ref, b_ref, o_ref):
    n = o_ref.shape[0]
    s = p_ref[0, :n, :] + p_ref[1, :n, :]
    o = dis_ref[:n, :] * (s + q_ref[:n, :])
    o_ref[...] = (jnp.dot(o, w_ref[...], preferred_element_type=jnp.float32)
                  + b_ref[...])


def kernel(x, edge_index, W1, b1, W2, b2):
    n, f_in = x.shape
    hid = W1.shape[1]
    c_out = W2.shape[1]
    e = edge_index.shape[1]

    per_dma = NW * CH
    nchunk = -(-e // per_dma)
    ep = nchunk * per_dma
    src = edge_index[0]
    dst = edge_index[1]
    padv = (n + jnp.arange(ep - e, dtype=jnp.int32) % (NPAD - n)
            ).astype(jnp.int32)
    srcw = jnp.concatenate([src, padv]).reshape(NW, nchunk, CH)
    dstw = jnp.concatenate([dst, padv]).reshape(NW, nchunk, CH)

    deg_k = _make_deg_kernel(nchunk)
    agg1_k = _make_agg1_kernel(nchunk)
    agg2_k = _make_agg2_kernel(nchunk)

    deg_parts = deg_k(dstw)

    mblk = 1000
    grid = (n // mblk,)
    h = pl.pallas_call(
        _matmul_kernel,
        grid=grid,
        in_specs=[
            pl.BlockSpec((mblk, f_in), lambda i: (i, 0)),
            pl.BlockSpec((f_in, hid), lambda i: (0, 0)),
        ],
        out_specs=pl.BlockSpec((mblk, hid), lambda i: (i, 0)),
        out_shape=jax.ShapeDtypeStruct((n, hid), jnp.float32),
    )(x, W1)
    hp = jnp.pad(h, ((0, NPAD - n), (0, 0)))

    s1_parts, g1p, dis_tp = agg1_k(hp, deg_parts, srcw, dstw)

    s2_parts, qp = agg2_k(g1p, s1_parts, dis_tp,
                          jnp.broadcast_to(b1, (1, hid)), srcw, dstw)

    cpad = -c_out % 8
    w2p = jnp.pad(W2, ((0, 0), (0, cpad)))
    b2p = jnp.pad(b2, (0, cpad))
    outp = pl.pallas_call(
        _final_kernel,
        out_shape=jax.ShapeDtypeStruct((n, c_out + cpad), jnp.float32),
    )(s2_parts, qp, dis_tp, w2p, jnp.broadcast_to(b2p, (1, c_out + cpad)))
    return outp[:, :c_out]
